# baseline (speedup 1.0000x reference)
import jax
import jax.numpy as jnp
from jax.experimental import pallas as pl
from jax.experimental.pallas import tpu as pltpu

_BLOCK_ROWS = 1024
_COL_SPLIT = 2


def _fused_lowrank_kernel(x_ref, win_ref, wout_ref, out_ref, h_ref):
    @pl.when(pl.program_id(1) == 0)
    def _compute_h():
        h_ref[...] = jnp.dot(x_ref[...], win_ref[...],
                             preferred_element_type=jnp.float32)

    out_ref[...] = jnp.dot(h_ref[...], wout_ref[...],
                           preferred_element_type=jnp.float32)


@jax.jit
def kernel(x, shared_in, shared_out):
    n_tokens, in_features = x.shape
    rank, out_features = shared_out.shape
    col_block = out_features // _COL_SPLIT

    grid = (n_tokens // _BLOCK_ROWS, _COL_SPLIT)
    return pl.pallas_call(
        _fused_lowrank_kernel,
        grid=grid,
        in_specs=[
            pl.BlockSpec((_BLOCK_ROWS, in_features), lambda i, j: (i, 0)),
            pl.BlockSpec((in_features, rank), lambda i, j: (0, 0)),
            pl.BlockSpec((rank, col_block), lambda i, j: (0, j)),
        ],
        out_specs=pl.BlockSpec((_BLOCK_ROWS, col_block), lambda i, j: (i, j)),
        out_shape=jax.ShapeDtypeStruct((n_tokens, out_features), jnp.float32),
        scratch_shapes=[pltpu.VMEM((_BLOCK_ROWS, rank), jnp.float32)],
        compiler_params=pltpu.CompilerParams(
            dimension_semantics=("arbitrary", "arbitrary"),
        ),
    )(x, shared_in, shared_out)


# speedup vs baseline: 1.4569x; 1.4569x over previous
import jax
import jax.numpy as jnp
from jax.experimental import pallas as pl
from jax.experimental.pallas import tpu as pltpu

_BLOCK_ROWS = 1024


def _fused_lowrank_kernel(x_ref, win_ref, wout_ref, out_ref):
    h = jnp.dot(x_ref[...], win_ref[...], preferred_element_type=jnp.float32)
    out_ref[...] = jnp.dot(h, wout_ref[...], preferred_element_type=jnp.float32)


@jax.jit
def kernel(x, shared_in, shared_out):
    n_tokens, in_features = x.shape
    rank, out_features = shared_out.shape

    grid = (n_tokens // _BLOCK_ROWS,)
    return pl.pallas_call(
        _fused_lowrank_kernel,
        grid=grid,
        in_specs=[
            pl.BlockSpec((_BLOCK_ROWS, in_features), lambda i: (i, 0)),
            pl.BlockSpec((in_features, rank), lambda i: (0, 0)),
            pl.BlockSpec((rank, out_features), lambda i: (0, 0)),
        ],
        out_specs=pl.BlockSpec((_BLOCK_ROWS, out_features), lambda i: (i, 0)),
        out_shape=jax.ShapeDtypeStruct((n_tokens, out_features), jnp.float32),
        compiler_params=pltpu.CompilerParams(
            dimension_semantics=("parallel",),
        ),
    )(x, shared_in, shared_out)


# speedup vs baseline: 1.6054x; 1.1020x over previous
import jax
import jax.numpy as jnp
from jax.experimental import pallas as pl
from jax.experimental.pallas import tpu as pltpu

_SUB = 512
_K = 4


def _fused_lowrank_kernel(x_hbm, win_ref, wout_ref, out_hbm,
                          xbuf, obuf, lsem, ssem):
    n_tokens = x_hbm.shape[0]
    n_steps = n_tokens // _SUB
    n_groups = n_steps // _K
    win = win_ref[...]
    wout = wout_ref[...]

    def load(step, slot):
        return pltpu.make_async_copy(
            x_hbm.at[pl.ds(step * _SUB, _SUB), :], xbuf.at[slot],
            lsem.at[slot])

    def store(step, slot):
        return pltpu.make_async_copy(
            obuf.at[slot], out_hbm.at[pl.ds(step * _SUB, _SUB), :],
            ssem.at[slot])

    def compute(slot):
        h = jnp.dot(xbuf[slot], win, preferred_element_type=jnp.float32)
        obuf[slot] = jnp.dot(h, wout, preferred_element_type=jnp.float32)

    for k in range(_K):
        load(k, k).start()

    for k in range(_K):
        load(k, k).wait()
        compute(k)
        store(k, k).start()
        load(k + _K, k).start()

    def group_body(g, carry):
        base = g * _K
        for k in range(_K):
            step = base + k
            load(step, k).wait()
            store(step - _K, k).wait()
            compute(k)
            store(step, k).start()
            load(step + _K, k).start()
        return carry

    jax.lax.fori_loop(1, n_groups - 1, group_body, 0)

    base = (n_groups - 1) * _K
    for k in range(_K):
        load(base + k, k).wait()
        store(base + k - _K, k).wait()
        compute(k)
        store(base + k, k).start()

    for k in range(_K):
        store(base + k, k).wait()


@jax.jit
def kernel(x, shared_in, shared_out):
    n_tokens, in_features = x.shape
    rank, out_features = shared_out.shape

    return pl.pallas_call(
        _fused_lowrank_kernel,
        in_specs=[
            pl.BlockSpec(memory_space=pltpu.MemorySpace.HBM),
            pl.BlockSpec(memory_space=pltpu.MemorySpace.VMEM),
            pl.BlockSpec(memory_space=pltpu.MemorySpace.VMEM),
        ],
        out_specs=pl.BlockSpec(memory_space=pltpu.MemorySpace.HBM),
        out_shape=jax.ShapeDtypeStruct((n_tokens, out_features), jnp.float32),
        scratch_shapes=[
            pltpu.VMEM((_K, _SUB, in_features), jnp.float32),
            pltpu.VMEM((_K, _SUB, out_features), jnp.float32),
            pltpu.SemaphoreType.DMA((_K,)),
            pltpu.SemaphoreType.DMA((_K,)),
        ],
    )(x, shared_in, shared_out)
